# Initial kernel scaffold; baseline (speedup 1.0000x reference)
import jax
import jax.numpy as jnp
from jax.experimental import pallas as pl


def kernel(x, edge_index, batch, W1, b1, W2, b2, fcW, fcb):
    raise NotImplementedError("write your pallas kernel here")


# speedup vs baseline: 23.2665x; 23.2665x over previous
import functools

import jax
import jax.numpy as jnp
from jax import lax
from jax.experimental import pallas as pl
from jax.experimental.pallas import tpu as pltpu
from jax.experimental.pallas import tpu_sc as plsc

NN = 100000
EE = 1600000
CC = 32
NB = 8

NPAD = 100352
EPAD = 1605632
NROWB = NPAD // 128
NBLK = 98
BLKN = 1024

NSC = 2
NTI = 16
SLICE = NPAD // NTI

_MESH = dict(core_axis_name="c", subcore_axis_name="s")


def _fill_ones(ref, n):
    def body(i, _):
        ref[pl.ds(i * 16, 16)] = jnp.ones((16,), jnp.float32)
        return 0
    lax.fori_loop(0, n // 16, body, 0)


def _fill_zero_rows(ref, nrows):
    def body(i, _):
        ref[i, :] = jnp.zeros((16,), jnp.float32)
        return 0
    lax.fori_loop(0, nrows, body, 0)


def _sc_deg(dst2):
    kern = pl.kernel(
        _sc_deg_body,
        out_type=jax.ShapeDtypeStruct((NSC, NPAD), jnp.float32),
        mesh=plsc.VectorSubcoreMesh(**_MESH),
        scratch_types=[
            pltpu.VMEM_SHARED((NPAD,), jnp.float32),
            pltpu.VMEM((8, 128), jnp.int32),
            pltpu.VMEM((128,), jnp.float32),
            pltpu.VMEM((SLICE,), jnp.float32),
        ],
    )
    return kern(dst2)


def _sc_deg_body(dst2_hbm, out_hbm, acc_sh, dbuf, ones_b, zbuf):
    cid = lax.axis_index("c")
    sid = lax.axis_index("s")
    _fill_ones(ones_b, 128)

    def zb(i, _):
        zbuf[pl.ds(i * 16, 16)] = jnp.zeros((16,), jnp.float32)
        return 0
    lax.fori_loop(0, SLICE // 16, zb, 0)
    pltpu.sync_copy(zbuf, acc_sh.at[pl.ds(sid * SLICE, SLICE)])
    plsc.subcore_barrier()

    nchunk = EPAD // (NSC * NTI) // 1024
    wrow0 = (cid * NTI + sid) * (nchunk * 8)

    def chunk(i, _):
        r0 = pl.multiple_of(wrow0 + i * 8, 8)
        pltpu.sync_copy(dst2_hbm.at[pl.ds(r0, 8)], dbuf)
        for j in range(8):
            pltpu.sync_copy(ones_b, acc_sh.at[dbuf.at[j]], add=True)
        return 0
    lax.fori_loop(0, nchunk, chunk, 0)
    plsc.subcore_barrier()
    pltpu.sync_copy(acc_sh.at[pl.ds(sid * SLICE, SLICE)],
                    out_hbm.at[cid].at[pl.ds(sid * SLICE, SLICE)])


def _sc_s1(src2, dst2, xs):
    kern = pl.kernel(
        _sc_s1_body,
        out_type=jax.ShapeDtypeStruct((NSC, NPAD), jnp.float32),
        mesh=plsc.VectorSubcoreMesh(**_MESH),
        scratch_types=[
            pltpu.VMEM_SHARED((NPAD,), jnp.float32),
            pltpu.VMEM((8, 128), jnp.int32),
            pltpu.VMEM((8, 128), jnp.int32),
            pltpu.VMEM((128,), jnp.float32),
            pltpu.VMEM((SLICE,), jnp.float32),
            pltpu.SemaphoreType.DMA,
        ],
    )
    return kern(src2, dst2, xs)


def _sc_s1_body(src2_hbm, dst2_hbm, xs_hbm, out_hbm,
                acc_sh, sbuf, dbuf, vbuf, zbuf, sem):
    cid = lax.axis_index("c")
    sid = lax.axis_index("s")

    def zb(i, _):
        zbuf[pl.ds(i * 16, 16)] = jnp.zeros((16,), jnp.float32)
        return 0
    lax.fori_loop(0, SLICE // 16, zb, 0)
    pltpu.sync_copy(zbuf, acc_sh.at[pl.ds(sid * SLICE, SLICE)])
    plsc.subcore_barrier()

    nchunk = EPAD // (NSC * NTI) // 1024
    wrow0 = (cid * NTI + sid) * (nchunk * 8)

    def chunk(i, _):
        r0 = pl.multiple_of(wrow0 + i * 8, 8)
        pltpu.sync_copy(src2_hbm.at[pl.ds(r0, 8)], sbuf)
        pltpu.sync_copy(dst2_hbm.at[pl.ds(r0, 8)], dbuf)
        for j in range(8):
            pltpu.async_copy(xs_hbm.at[sbuf.at[j]], vbuf, sem).wait()
            pltpu.sync_copy(vbuf, acc_sh.at[dbuf.at[j]], add=True)
        return 0
    lax.fori_loop(0, nchunk, chunk, 0)
    plsc.subcore_barrier()
    pltpu.sync_copy(acc_sh.at[pl.ds(sid * SLICE, SLICE)],
                    out_hbm.at[cid].at[pl.ds(sid * SLICE, SLICE)])


def _sc_q(src1, dst2, rs):
    kern = pl.kernel(
        _sc_q_body,
        out_type=jax.ShapeDtypeStruct((NSC, NPAD, 16), jnp.float32),
        mesh=plsc.VectorSubcoreMesh(**_MESH),
        compiler_params=pltpu.CompilerParams(use_tc_tiling_on_sc=False),
        scratch_types=[
            pltpu.VMEM_SHARED((NPAD, 16), jnp.float32),
            pltpu.VMEM((1024,), jnp.int32),
            pltpu.VMEM((8, 128), jnp.int32),
            pltpu.VMEM((8, 128, 16), jnp.float32),
            pltpu.VMEM((392, 16), jnp.float32),
            pltpu.SemaphoreType.DMA,
        ],
    )
    return kern(src1, dst2, rs)


def _sc_q_body(src1_hbm, dst2_hbm, rs_hbm, out_hbm,
               acc_sh, sbuf, dbuf, rows, zbuf, sem):
    cid = lax.axis_index("c")
    sid = lax.axis_index("s")
    _fill_zero_rows(zbuf, 392)
    for t in range(16):
        pltpu.sync_copy(zbuf, acc_sh.at[pl.ds(sid * SLICE + t * 392, 392)])
    plsc.subcore_barrier()

    off = cid * NPAD
    nchunk = EPAD // NTI // 1024
    ebase0 = sid * (nchunk * 1024)

    def chunk(i, _):
        ebase = pl.multiple_of(ebase0 + i * 1024, 1024)
        pltpu.sync_copy(src1_hbm.at[pl.ds(ebase, 1024)], sbuf)
        pltpu.sync_copy(dst2_hbm.at[pl.ds(pl.multiple_of(ebase0 // 128 + i * 8, 8), 8)], dbuf)

        def addoff(k, _2):
            sbuf[pl.ds(k * 16, 16)] = sbuf[pl.ds(k * 16, 16)] + off
            return 0
        lax.fori_loop(0, 64, addoff, 0)
        for j in range(8):
            pltpu.async_copy(rs_hbm.at[sbuf.at[pl.ds(j * 128, 128)]],
                             rows.at[j], sem).wait()
            pltpu.sync_copy(rows.at[j], acc_sh.at[dbuf.at[j]], add=True)
        return 0
    lax.fori_loop(0, nchunk, chunk, 0)
    plsc.subcore_barrier()
    pltpu.sync_copy(acc_sh.at[pl.ds(sid * SLICE, SLICE)],
                    out_hbm.at[cid].at[pl.ds(sid * SLICE, SLICE)])


def _tc_b_body(dega, degb, x2, dinv_o, xs_o):
    deg = dega[...] + degb[...] + 1.0
    dinv = lax.rsqrt(jnp.maximum(deg, 1.0))
    dinv_o[...] = dinv
    xs_o[...] = dinv * x2[...]


def _tc_b(dega, degb, x2):
    return pl.pallas_call(
        _tc_b_body,
        out_shape=[jax.ShapeDtypeStruct((NROWB, 128), jnp.float32)] * 2,
    )(dega, degb, x2)


def _tc_d_body(s1a, s1b, dinv, xc, w1, b1, agg_o, r1_o, rs_o):
    d = dinv[...]
    s1 = s1a[...] + s1b[...]
    agg = d * s1 + d * d * xc[...]
    agg_o[...] = agg
    r1 = jnp.maximum(agg * w1[...] + b1[...], 0.0)
    r1_o[...] = r1
    rs = r1 * d
    rs_o[0, :, :] = rs[:, :16]
    rs_o[1, :, :] = rs[:, 16:]


def _tc_d(s1a, s1b, dinvc, xc, w1, b1):
    col = pl.BlockSpec((BLKN, 1), lambda i: (i, 0))
    return pl.pallas_call(
        _tc_d_body,
        grid=(NBLK,),
        in_specs=[col, col, col, col,
                  pl.BlockSpec((1, CC), lambda i: (0, 0)),
                  pl.BlockSpec((1, CC), lambda i: (0, 0))],
        out_specs=[col,
                   pl.BlockSpec((BLKN, CC), lambda i: (i, 0)),
                   pl.BlockSpec((2, BLKN, 16), lambda i: (0, i, 0))],
        out_shape=[jax.ShapeDtypeStruct((NPAD, 1), jnp.float32),
                   jax.ShapeDtypeStruct((NPAD, CC), jnp.float32),
                   jax.ShapeDtypeStruct((2, NPAD, 16), jnp.float32)],
    )(s1a, s1b, dinvc, xc, w1, b1)


def _tc_f_body(q2, r1, agg, dinv, xc, bc, w2, b2, fcw, fcb,
               out_o, sums_o, cnt_o):
    i = pl.program_id(0)
    d = dinv[...]
    q = jnp.concatenate([q2[0], q2[1]], axis=1)
    r1v = r1[...]
    m = agg[...] + d * q + d * d * r1v
    g = jnp.dot(m, w2[...], preferred_element_type=jnp.float32) + b2[...]
    h2 = xc[...] + r1v + jnp.maximum(g, 0.0)
    oh = (bc[...] == lax.broadcasted_iota(jnp.int32, (BLKN, NB), 1))
    oh = oh.astype(jnp.float32)
    ps = lax.dot_general(oh, h2, (((0,), (0,)), ((), ())),
                         preferred_element_type=jnp.float32)
    ones = jnp.ones((BLKN, 1), jnp.float32)
    pc = lax.dot_general(oh, ones, (((0,), (0,)), ((), ())),
                         preferred_element_type=jnp.float32)

    @pl.when(i == 0)
    def _():
        sums_o[...] = jnp.zeros_like(sums_o)
        cnt_o[...] = jnp.zeros_like(cnt_o)
        out_o[...] = jnp.zeros_like(out_o)

    sums_o[...] += ps
    cnt_o[...] += pc

    @pl.when(i == NBLK - 1)
    def _():
        pooled = sums_o[...] / jnp.maximum(cnt_o[...], 1.0)
        z = jnp.dot(pooled, fcw[...],
                    preferred_element_type=jnp.float32) + fcb[...]
        out_o[...] = 1.0 / (1.0 + jnp.exp(-z))


def _tc_f(q2, r1, aggc, dinvc, xc, bc, w2, b2, fcw, fcb):
    col = pl.BlockSpec((BLKN, 1), lambda i: (i, 0))
    fixed = lambda shape: pl.BlockSpec(shape, lambda i: tuple(0 for _ in shape))
    return pl.pallas_call(
        _tc_f_body,
        grid=(NBLK,),
        in_specs=[pl.BlockSpec((2, BLKN, 16), lambda i: (0, i, 0)),
                  pl.BlockSpec((BLKN, CC), lambda i: (i, 0)),
                  col, col, col, col,
                  fixed((CC, CC)), fixed((1, CC)),
                  fixed((CC, 1)), fixed((1, 1))],
        out_specs=[fixed((NB, 1)), fixed((NB, CC)), fixed((NB, 1))],
        out_shape=[jax.ShapeDtypeStruct((NB, 1), jnp.float32),
                   jax.ShapeDtypeStruct((NB, CC), jnp.float32),
                   jax.ShapeDtypeStruct((NB, 1), jnp.float32)],
    )(q2, r1, aggc, dinvc, xc, bc, w2, b2, fcw, fcb)


def kernel(x, edge_index, batch, W1, b1, W2, b2, fcW, fcb):
    src = edge_index[0].astype(jnp.int32)
    dst = edge_index[1].astype(jnp.int32)
    npad_e = EPAD - EE
    src_p = jnp.concatenate([src, jnp.zeros((npad_e,), jnp.int32)])
    dst_p = jnp.concatenate(
        [dst, jnp.full((npad_e,), NPAD - 1, jnp.int32)])
    dst2 = dst_p.reshape(EPAD // 128, 128)
    src2 = src_p.reshape(EPAD // 128, 128)

    xpad = jnp.pad(x[:, 0], (0, NPAD - NN))
    bpad = jnp.pad(batch.astype(jnp.int32), (0, NPAD - NN),
                   constant_values=NB)

    deg2 = _sc_deg(dst2)
    dinv2d, xs2d = _tc_b(deg2[0].reshape(NROWB, 128),
                         deg2[1].reshape(NROWB, 128),
                         xpad.reshape(NROWB, 128))
    xs = xs2d.reshape(NPAD)
    s12 = _sc_s1(src2, dst2, xs)

    dinvc = dinv2d.reshape(NPAD, 1)
    xc = xpad.reshape(NPAD, 1)
    aggc, r1, rs2 = _tc_d(s12[0].reshape(NPAD, 1), s12[1].reshape(NPAD, 1),
                          dinvc, xc, W1, b1.reshape(1, CC))

    q2 = _sc_q(src_p, dst2, rs2.reshape(2 * NPAD, 16))

    out, _, _ = _tc_f(q2, r1, aggc, dinvc, xc, bpad.reshape(NPAD, 1),
                      W2, b2.reshape(1, CC), fcW, fcb.reshape(1, 1))
    return out


# speedup vs baseline: 40.6900x; 1.7489x over previous
import functools

import jax
import jax.numpy as jnp
from jax import lax
from jax.experimental import pallas as pl
from jax.experimental.pallas import tpu as pltpu
from jax.experimental.pallas import tpu_sc as plsc

NN = 100000
EE = 1600000
CC = 32
NB = 8

NPAD = 100352
EPAD = 1605632
NROWB = NPAD // 128
NBLK = 98
BLKN = 1024

NSC = 2
NTI = 16
SLICE = NPAD // NTI

_MESH = dict(core_axis_name="c", subcore_axis_name="s")


def _fill_ones(ref, n):
    def body(i, _):
        ref[pl.ds(i * 16, 16)] = jnp.ones((16,), jnp.float32)
        return 0
    lax.fori_loop(0, n // 16, body, 0)


def _fill_zero_rows(ref, nrows):
    def body(i, _):
        ref[i, :] = jnp.zeros((16,), jnp.float32)
        return 0
    lax.fori_loop(0, nrows, body, 0)


def _sc_deg(dst2):
    kern = pl.kernel(
        _sc_deg_body,
        out_type=jax.ShapeDtypeStruct((NSC, NPAD), jnp.float32),
        mesh=plsc.VectorSubcoreMesh(**_MESH),
        scratch_types=[
            pltpu.VMEM_SHARED((NPAD,), jnp.float32),
            pltpu.VMEM((8, 128), jnp.int32),
            pltpu.VMEM((8, 128), jnp.int32),
            pltpu.VMEM((128,), jnp.float32),
            pltpu.VMEM((SLICE,), jnp.float32),
            pltpu.SemaphoreType.DMA,
            pltpu.SemaphoreType.DMA,
        ],
    )
    return kern(dst2)


def _sc_deg_body(dst2_hbm, out_hbm, acc_sh, dbuf0, dbuf1, ones_b, zbuf,
                 isem, ssem):
    cid = lax.axis_index("c")
    sid = lax.axis_index("s")
    _fill_ones(ones_b, 128)

    def zb(i, _):
        zbuf[pl.ds(i * 16, 16)] = jnp.zeros((16,), jnp.float32)
        return 0
    lax.fori_loop(0, SLICE // 16, zb, 0)
    pltpu.sync_copy(zbuf, acc_sh.at[pl.ds(sid * SLICE, SLICE)])
    plsc.subcore_barrier()

    nchunk = EPAD // (NSC * NTI) // 1024
    wrow0 = (cid * NTI + sid) * (nchunk * 8)
    dbufs = (dbuf0, dbuf1)

    def pair(k, _):
        idx = []
        for b in (0, 1):
            r0 = pl.multiple_of(wrow0 + (2 * k + b) * 8, 8)
            idx.append(pltpu.async_copy(dst2_hbm.at[pl.ds(r0, 8)],
                                        dbufs[b], isem))
        sd = []
        for b in (0, 1):
            idx[b].wait()
            for j in range(8):
                sd.append(pltpu.async_copy(
                    ones_b, acc_sh.at[dbufs[b].at[j]], ssem, add=True))
        for d in sd:
            d.wait()
        return 0
    lax.fori_loop(0, nchunk // 2, pair, 0)
    r0 = pl.multiple_of(wrow0 + (nchunk - 1) * 8, 8)
    pltpu.sync_copy(dst2_hbm.at[pl.ds(r0, 8)], dbuf0)
    sd = []
    for j in range(8):
        sd.append(pltpu.async_copy(ones_b, acc_sh.at[dbuf0.at[j]],
                                   ssem, add=True))
    for d in sd:
        d.wait()
    plsc.subcore_barrier()
    pltpu.sync_copy(acc_sh.at[pl.ds(sid * SLICE, SLICE)],
                    out_hbm.at[cid].at[pl.ds(sid * SLICE, SLICE)])


def _sc_s1(src2, dst2, xs):
    kern = pl.kernel(
        _sc_s1_body,
        out_type=jax.ShapeDtypeStruct((NSC, NPAD), jnp.float32),
        mesh=plsc.VectorSubcoreMesh(**_MESH),
        scratch_types=[
            pltpu.VMEM_SHARED((NPAD,), jnp.float32),
            pltpu.VMEM((8, 128), jnp.int32),
            pltpu.VMEM((8, 128), jnp.int32),
            pltpu.VMEM((8, 128), jnp.int32),
            pltpu.VMEM((8, 128), jnp.int32),
            pltpu.VMEM((8, 128), jnp.float32),
            pltpu.VMEM((8, 128), jnp.float32),
            pltpu.VMEM((SLICE,), jnp.float32),
            pltpu.SemaphoreType.DMA,
            pltpu.SemaphoreType.DMA,
            pltpu.SemaphoreType.DMA,
        ],
    )
    return kern(src2, dst2, xs)


def _sc_s1_body(src2_hbm, dst2_hbm, xs_hbm, out_hbm,
                acc_sh, sbuf0, sbuf1, dbuf0, dbuf1, vbuf0, vbuf1, zbuf,
                isem, gsem, ssem):
    cid = lax.axis_index("c")
    sid = lax.axis_index("s")

    def zb(i, _):
        zbuf[pl.ds(i * 16, 16)] = jnp.zeros((16,), jnp.float32)
        return 0
    lax.fori_loop(0, SLICE // 16, zb, 0)
    pltpu.sync_copy(zbuf, acc_sh.at[pl.ds(sid * SLICE, SLICE)])
    plsc.subcore_barrier()

    nchunk = EPAD // (NSC * NTI) // 1024
    wrow0 = (cid * NTI + sid) * (nchunk * 8)
    bufs = ((sbuf0, dbuf0, vbuf0), (sbuf1, dbuf1, vbuf1))

    def chunk_gathers(ii, b):
        sb, db, vb = bufs[b]
        r0 = pl.multiple_of(wrow0 + ii * 8, 8)
        i1 = pltpu.async_copy(src2_hbm.at[pl.ds(r0, 8)], sb, isem)
        i2 = pltpu.async_copy(dst2_hbm.at[pl.ds(r0, 8)], db, isem)
        i1.wait()
        i2.wait()
        return [pltpu.async_copy(xs_hbm.at[sb.at[j]], vb.at[j], gsem)
                for j in range(8)]

    def chunk_scatters(gd, b):
        sb, db, vb = bufs[b]
        sd = []
        for j in range(8):
            gd[j].wait()
            sd.append(pltpu.async_copy(vb.at[j], acc_sh.at[db.at[j]],
                                       ssem, add=True))
        return sd

    def pair(k, _):
        gd0 = chunk_gathers(2 * k, 0)
        gd1 = chunk_gathers(2 * k + 1, 1)
        sd = chunk_scatters(gd0, 0) + chunk_scatters(gd1, 1)
        for d in sd:
            d.wait()
        return 0
    lax.fori_loop(0, nchunk // 2, pair, 0)
    gd0 = chunk_gathers(nchunk - 1, 0)
    for d in chunk_scatters(gd0, 0):
        d.wait()
    plsc.subcore_barrier()
    pltpu.sync_copy(acc_sh.at[pl.ds(sid * SLICE, SLICE)],
                    out_hbm.at[cid].at[pl.ds(sid * SLICE, SLICE)])


def _sc_q(src1, dst2, rs):
    kern = pl.kernel(
        _sc_q_body,
        out_type=jax.ShapeDtypeStruct((NSC, NPAD, 16), jnp.float32),
        mesh=plsc.VectorSubcoreMesh(**_MESH),
        compiler_params=pltpu.CompilerParams(use_tc_tiling_on_sc=False),
        scratch_types=[
            pltpu.VMEM_SHARED((NPAD, 16), jnp.float32),
            pltpu.VMEM((1024,), jnp.int32),
            pltpu.VMEM((8, 128), jnp.int32),
            pltpu.VMEM((4, 128, 16), jnp.float32),
            pltpu.VMEM((4, 128, 16), jnp.float32),
            pltpu.SemaphoreType.DMA,
            pltpu.SemaphoreType.DMA,
            pltpu.SemaphoreType.DMA,
        ],
    )
    return kern(src1, dst2, rs)


def _sc_q_body(src1_hbm, dst2_hbm, rs_hbm, out_hbm,
               acc_sh, sbuf, dbuf, rows0, rows1,
               isem, gsem, ssem):
    cid = lax.axis_index("c")
    sid = lax.axis_index("s")
    _fill_zero_rows(rows0.at[0], 128)
    for t in range(SLICE // 128):
        pltpu.sync_copy(rows0.at[0],
                        acc_sh.at[pl.ds(sid * SLICE + t * 128, 128)])
    plsc.subcore_barrier()

    off = cid * NPAD
    nchunk = EPAD // NTI // 1024
    ebase0 = sid * (nchunk * 1024)
    rowbufs = (rows0, rows1)

    def chunk(k, _):
        eb = pl.multiple_of(ebase0 + k * 1024, 1024)
        rb = pl.multiple_of(ebase0 // 128 + k * 8, 8)
        i1 = pltpu.async_copy(src1_hbm.at[pl.ds(eb, 1024)], sbuf, isem)
        i2 = pltpu.async_copy(dst2_hbm.at[pl.ds(rb, 8)], dbuf, isem)
        i1.wait()
        i2.wait()

        def addoff(kk, _2):
            sbuf[pl.ds(kk * 16, 16)] = sbuf[pl.ds(kk * 16, 16)] + off
            return 0
        lax.fori_loop(0, 64, addoff, 0)
        gd = [pltpu.async_copy(rs_hbm.at[sbuf.at[pl.ds(j * 128, 128)]],
                               rowbufs[j // 4].at[j % 4], gsem)
              for j in range(8)]
        sd = []
        for j in range(8):
            gd[j].wait()
            sd.append(pltpu.async_copy(rowbufs[j // 4].at[j % 4],
                                       acc_sh.at[dbuf.at[j]],
                                       ssem, add=True))
        for d in sd:
            d.wait()
        return 0
    lax.fori_loop(0, nchunk, chunk, 0)
    plsc.subcore_barrier()
    pltpu.sync_copy(acc_sh.at[pl.ds(sid * SLICE, SLICE)],
                    out_hbm.at[cid].at[pl.ds(sid * SLICE, SLICE)])


def _tc_b_body(dega, degb, x2, dinv_o, xs_o):
    deg = dega[...] + degb[...] + 1.0
    dinv = lax.rsqrt(jnp.maximum(deg, 1.0))
    dinv_o[...] = dinv
    xs_o[...] = dinv * x2[...]


def _tc_b(dega, degb, x2):
    return pl.pallas_call(
        _tc_b_body,
        out_shape=[jax.ShapeDtypeStruct((NROWB, 128), jnp.float32)] * 2,
    )(dega, degb, x2)


def _tc_d_body(s1a, s1b, dinv, xc, w1, b1, agg_o, r1_o, rs_o):
    d = dinv[...]
    s1 = s1a[...] + s1b[...]
    agg = d * s1 + d * d * xc[...]
    agg_o[...] = agg
    r1 = jnp.maximum(agg * w1[...] + b1[...], 0.0)
    r1_o[...] = r1
    rs = r1 * d
    rs_o[0, :, :] = rs[:, :16]
    rs_o[1, :, :] = rs[:, 16:]


def _tc_d(s1a, s1b, dinvc, xc, w1, b1):
    col = pl.BlockSpec((BLKN, 1), lambda i: (i, 0))
    return pl.pallas_call(
        _tc_d_body,
        grid=(NBLK,),
        in_specs=[col, col, col, col,
                  pl.BlockSpec((1, CC), lambda i: (0, 0)),
                  pl.BlockSpec((1, CC), lambda i: (0, 0))],
        out_specs=[col,
                   pl.BlockSpec((BLKN, CC), lambda i: (i, 0)),
                   pl.BlockSpec((2, BLKN, 16), lambda i: (0, i, 0))],
        out_shape=[jax.ShapeDtypeStruct((NPAD, 1), jnp.float32),
                   jax.ShapeDtypeStruct((NPAD, CC), jnp.float32),
                   jax.ShapeDtypeStruct((2, NPAD, 16), jnp.float32)],
    )(s1a, s1b, dinvc, xc, w1, b1)


def _tc_f_body(q2, r1, agg, dinv, xc, bc, w2, b2, fcw, fcb,
               out_o, sums_o, cnt_o):
    i = pl.program_id(0)
    d = dinv[...]
    q = jnp.concatenate([q2[0], q2[1]], axis=1)
    r1v = r1[...]
    m = agg[...] + d * q + d * d * r1v
    g = jnp.dot(m, w2[...], preferred_element_type=jnp.float32) + b2[...]
    h2 = xc[...] + r1v + jnp.maximum(g, 0.0)
    oh = (bc[...] == lax.broadcasted_iota(jnp.int32, (BLKN, NB), 1))
    oh = oh.astype(jnp.float32)
    ps = lax.dot_general(oh, h2, (((0,), (0,)), ((), ())),
                         preferred_element_type=jnp.float32)
    ones = jnp.ones((BLKN, 1), jnp.float32)
    pc = lax.dot_general(oh, ones, (((0,), (0,)), ((), ())),
                         preferred_element_type=jnp.float32)

    @pl.when(i == 0)
    def _():
        sums_o[...] = jnp.zeros_like(sums_o)
        cnt_o[...] = jnp.zeros_like(cnt_o)
        out_o[...] = jnp.zeros_like(out_o)

    sums_o[...] += ps
    cnt_o[...] += pc

    @pl.when(i == NBLK - 1)
    def _():
        pooled = sums_o[...] / jnp.maximum(cnt_o[...], 1.0)
        z = jnp.dot(pooled, fcw[...],
                    preferred_element_type=jnp.float32) + fcb[...]
        out_o[...] = 1.0 / (1.0 + jnp.exp(-z))


def _tc_f(q2, r1, aggc, dinvc, xc, bc, w2, b2, fcw, fcb):
    col = pl.BlockSpec((BLKN, 1), lambda i: (i, 0))
    fixed = lambda shape: pl.BlockSpec(shape, lambda i: tuple(0 for _ in shape))
    return pl.pallas_call(
        _tc_f_body,
        grid=(NBLK,),
        in_specs=[pl.BlockSpec((2, BLKN, 16), lambda i: (0, i, 0)),
                  pl.BlockSpec((BLKN, CC), lambda i: (i, 0)),
                  col, col, col, col,
                  fixed((CC, CC)), fixed((1, CC)),
                  fixed((CC, 1)), fixed((1, 1))],
        out_specs=[fixed((NB, 1)), fixed((NB, CC)), fixed((NB, 1))],
        out_shape=[jax.ShapeDtypeStruct((NB, 1), jnp.float32),
                   jax.ShapeDtypeStruct((NB, CC), jnp.float32),
                   jax.ShapeDtypeStruct((NB, 1), jnp.float32)],
    )(q2, r1, aggc, dinvc, xc, bc, w2, b2, fcw, fcb)


def kernel(x, edge_index, batch, W1, b1, W2, b2, fcW, fcb):
    src = edge_index[0].astype(jnp.int32)
    dst = edge_index[1].astype(jnp.int32)
    npad_e = EPAD - EE
    src_p = jnp.concatenate([src, jnp.zeros((npad_e,), jnp.int32)])
    dst_p = jnp.concatenate(
        [dst, jnp.full((npad_e,), NPAD - 1, jnp.int32)])
    dst2 = dst_p.reshape(EPAD // 128, 128)
    src2 = src_p.reshape(EPAD // 128, 128)

    xpad = jnp.pad(x[:, 0], (0, NPAD - NN))
    bpad = jnp.pad(batch.astype(jnp.int32), (0, NPAD - NN),
                   constant_values=NB)

    deg2 = _sc_deg(dst2)
    dinv2d, xs2d = _tc_b(deg2[0].reshape(NROWB, 128),
                         deg2[1].reshape(NROWB, 128),
                         xpad.reshape(NROWB, 128))
    xs = xs2d.reshape(NPAD)
    s12 = _sc_s1(src2, dst2, xs)

    dinvc = dinv2d.reshape(NPAD, 1)
    xc = xpad.reshape(NPAD, 1)
    aggc, r1, rs2 = _tc_d(s12[0].reshape(NPAD, 1), s12[1].reshape(NPAD, 1),
                          dinvc, xc, W1, b1.reshape(1, CC))

    q2 = _sc_q(src_p, dst2, rs2.reshape(2 * NPAD, 16))

    out, _, _ = _tc_f(q2, r1, aggc, dinvc, xc, bpad.reshape(NPAD, 1),
                      W2, b2.reshape(1, CC), fcW, fcb.reshape(1, 1))
    return out
